# baseline (speedup 1.0000x reference)
import functools

import jax
import jax.numpy as jnp
from jax import lax
from jax.experimental import pallas as pl
from jax.experimental.pallas import tpu as pltpu
from jax.experimental.pallas import tpu_sc as plsc

_N = 50000
_E = 800000
_H = 64
_HH = 32
_CHUNK = 128
_TILES = 16
_CPT = 392
_EPAD = _TILES * _CPT * _CHUNK
_NROWS = _EPAD // _CHUNK
_NBUF = 4
_NP = 51200
_ZB = _NP // (_TILES * _CHUNK)
_PK = 12544
_PKT = _PK // _TILES
_R = 2048


def _sc_aggregate():
  mesh = plsc.VectorSubcoreMesh(core_axis_name="c", subcore_axis_name="s")
  out_type = [jax.ShapeDtypeStruct((2, 4 * _PK, _HH), jnp.float32),
              jax.ShapeDtypeStruct((_N,), jnp.float32)]
  scratch = [
      pltpu.VMEM((8, _CHUNK), jnp.int32),
      pltpu.VMEM((8, _CHUNK), jnp.int32),
      pltpu.VMEM((_NBUF, _CHUNK, _HH), jnp.float32),
      pltpu.VMEM_SHARED((_NP, _HH), jnp.float32),
      pltpu.SemaphoreType.DMA,
      pltpu.SemaphoreType.DMA,
      pltpu.SemaphoreType.DMA,
      pltpu.VMEM((_CHUNK,), jnp.float32),
      pltpu.VMEM((3200,), jnp.float32),
      pltpu.VMEM_SHARED((_NP,), jnp.float32),
  ]

  def body(h_lo, h_hi, srcr, dstr, agg_out, deg_out, src_v, dst_v, rows_v,
           agg_sh, sem_i, sem_g, sem_s, ones_v, dzero_v, deg_sh):
    c = lax.axis_index("c")
    s = lax.axis_index("s")

    def _zrow(i, _):
      rows_v[0, i, pl.ds(0, 16)] = jnp.zeros((16,), jnp.float32)
      rows_v[0, i, pl.ds(16, 16)] = jnp.zeros((16,), jnp.float32)
      return 0
    lax.fori_loop(0, _CHUNK, _zrow, 0)

    def _zshared(j, _):
      pltpu.sync_copy(rows_v.at[0],
                      agg_sh.at[pl.ds((s * _ZB + j) * _CHUNK, _CHUNK)])
      return 0
    lax.fori_loop(0, _ZB, _zshared, 0)

    def _zd(i, _):
      dzero_v[pl.ds(i * 16, 16)] = jnp.zeros((16,), jnp.float32)
      return 0
    lax.fori_loop(0, 200, _zd, 0)
    pltpu.sync_copy(dzero_v, deg_sh.at[pl.ds(s * 3200, 3200)])

    def _ones(i, _):
      ones_v[pl.ds(i * 16, 16)] = jnp.ones((16,), jnp.float32)
      return 0
    lax.fori_loop(0, _CHUNK // 16, _ones, 0)

    plsc.subcore_barrier()

    row0 = s * _CPT

    def issue_idx(j, b):
      pltpu.async_copy(srcr.at[row0 + j], src_v.at[b], sem_i)
      pltpu.async_copy(dstr.at[row0 + j], dst_v.at[b], sem_i)

    def issue_gather(ib, rb):
      @pl.when(c == 0)
      def _():
        pltpu.async_copy(h_lo.at[src_v.at[ib]], rows_v.at[rb], sem_g)

      @pl.when(c == 1)
      def _():
        pltpu.async_copy(h_hi.at[src_v.at[ib]], rows_v.at[rb], sem_g)

    def wait_idx_pair():
      pltpu.make_async_copy(dstr.at[0], src_v.at[0], sem_i).wait()
      pltpu.make_async_copy(dstr.at[0], dst_v.at[0], sem_i).wait()

    def wait_chunk(sem):
      pltpu.make_async_copy(h_lo.at[pl.ds(0, _CHUNK)], rows_v.at[0],
                            sem).wait()

    for b in range(4):
      issue_idx(b, b)
    wait_idx_pair()
    wait_idx_pair()
    issue_gather(0, 0)
    issue_gather(1, 1)

    def grp(g, _):
      for b in range(8):
        j = g * 8 + b

        @pl.when(j + 4 < _CPT)
        def _():
          issue_idx(j + 4, (b + 4) % 8)

        @pl.when(j >= 2)
        def _():
          wait_chunk(sem_s)

        @pl.when(j + 2 < _CPT)
        def _():
          wait_idx_pair()
          issue_gather((b + 2) % 8, (b + 2) % _NBUF)

        wait_chunk(sem_g)
        pltpu.async_copy(rows_v.at[b % _NBUF], agg_sh.at[dst_v.at[b % 8]],
                         sem_s, add=True)

        @pl.when(c == 0)
        def _():
          pltpu.sync_copy(ones_v, deg_sh.at[dst_v.at[b % 8]], add=True)
      return 0
    lax.fori_loop(0, _CPT // 8, grp, 0)

    wait_chunk(sem_s)
    wait_chunk(sem_s)

    plsc.subcore_barrier()

    pltpu.sync_copy(agg_sh.at[pl.ds(s * 4 * _PKT, 4 * _PKT)],
                    agg_out.at[c, pl.ds(s * 4 * _PKT, 4 * _PKT)])

    @pl.when(jnp.logical_and(c == 0, s == 0))
    def _():
      pltpu.sync_copy(deg_sh.at[pl.ds(0, _N)], deg_out)

  return pl.kernel(
      body, out_type=out_type, mesh=mesh, scratch_types=scratch,
      compiler_params=pltpu.CompilerParams(use_tc_tiling_on_sc=False))


_sc_agg_deg = _sc_aggregate()


def _full(shape):
  return pl.BlockSpec(shape, lambda i: tuple(0 for _ in shape))


def _tc_pre(x, cc, w1x, w1c, b1, wc, bc):
  def body(x_r, c_r, w1x_r, w1c_r, b1_r, wc_r, bc_r, lo_r, hi_r, c1_r):
    h = jnp.maximum(
        jnp.dot(x_r[...], w1x_r[...], preferred_element_type=jnp.float32)
        + jnp.dot(c_r[...], w1c_r[...], preferred_element_type=jnp.float32)
        + b1_r[...], 0.0)
    c1 = jnp.maximum(
        jnp.dot(c_r[...], wc_r[...], preferred_element_type=jnp.float32)
        + bc_r[...], 0.0)
    lo_r[...] = h[:, :_HH]
    hi_r[...] = h[:, _HH:]
    c1_r[...] = c1

  hh = pl.BlockSpec((_R, _HH), lambda i: (i, 0))
  return pl.pallas_call(
      body,
      grid=((_N + _R - 1) // _R,),
      in_specs=[
          pl.BlockSpec((_R, 128), lambda i: (i, 0)),
          pl.BlockSpec((_R, 16), lambda i: (i, 0)),
          _full((128, _H)), _full((16, _H)), _full((1, _H)),
          _full((16, _H)), _full((1, _H)),
      ],
      out_specs=[hh, hh, pl.BlockSpec((_R, _H), lambda i: (i, 0))],
      out_shape=[
          jax.ShapeDtypeStruct((_N, _HH), jnp.float32),
          jax.ShapeDtypeStruct((_N, _HH), jnp.float32),
          jax.ShapeDtypeStruct((_N, _H), jnp.float32),
      ],
  )(x, cc, w1x, w1c, b1, wc, bc)


def _tc_mid(hlo, hhi, agg, deg, c1, w2, b2, w1a, w1b, b11):
  def body(hlo_r, hhi_r, alo_r, ahi_r, deg_r, c1_r, w2_r, b2_r, w1a_r, w1b_r,
           b11_r, lo_r, hi_r):
    inv = 1.0 / jnp.maximum(deg_r[...], 1.0)
    u = jnp.concatenate(
        [hlo_r[...] + alo_r[0] * inv, hhi_r[...] + ahi_r[0] * inv], axis=1)
    x1 = jnp.dot(u, w2_r[...], preferred_element_type=jnp.float32) + b2_r[...]
    h1 = jnp.maximum(
        jnp.dot(x1, w1a_r[...], preferred_element_type=jnp.float32)
        + jnp.dot(c1_r[...], w1b_r[...], preferred_element_type=jnp.float32)
        + b11_r[...], 0.0)
    lo_r[...] = h1[:, :_HH]
    hi_r[...] = h1[:, _HH:]

  hh = pl.BlockSpec((_R, _HH), lambda i: (i, 0))
  alo = pl.BlockSpec((1, _R, _HH), lambda i: (0, i, 0))
  ahi = pl.BlockSpec((1, _R, _HH), lambda i: (1, i, 0))
  return pl.pallas_call(
      body,
      grid=((_N + _R - 1) // _R,),
      in_specs=[
          hh, hh, alo, ahi,
          pl.BlockSpec((_R, 1), lambda i: (i, 0)),
          pl.BlockSpec((_R, _H), lambda i: (i, 0)),
          _full((_H, _H)), _full((1, _H)),
          _full((_H, _H)), _full((_H, _H)), _full((1, _H)),
      ],
      out_specs=[hh, hh],
      out_shape=[
          jax.ShapeDtypeStruct((_N, _HH), jnp.float32),
          jax.ShapeDtypeStruct((_N, _HH), jnp.float32),
      ],
  )(hlo, hhi, agg, agg, deg, c1, w2, b2, w1a, w1b, b11)


def _tc_fin(hlo, hhi, agg, deg, w2, b2):
  def body(hlo_r, hhi_r, alo_r, ahi_r, deg_r, w2_r, b2_r, out_r):
    inv = 1.0 / jnp.maximum(deg_r[...], 1.0)
    u = jnp.concatenate(
        [hlo_r[...] + alo_r[0] * inv, hhi_r[...] + ahi_r[0] * inv], axis=1)
    out_r[...] = (
        jnp.dot(u, w2_r[...], preferred_element_type=jnp.float32) + b2_r[...])

  hh = pl.BlockSpec((_R, _HH), lambda i: (i, 0))
  alo = pl.BlockSpec((1, _R, _HH), lambda i: (0, i, 0))
  ahi = pl.BlockSpec((1, _R, _HH), lambda i: (1, i, 0))
  return pl.pallas_call(
      body,
      grid=((_N + _R - 1) // _R,),
      in_specs=[
          hh, hh, alo, ahi,
          pl.BlockSpec((_R, 1), lambda i: (i, 0)),
          _full((_H, 128)), _full((1, 128)),
      ],
      out_specs=pl.BlockSpec((_R, 128), lambda i: (i, 0)),
      out_shape=jax.ShapeDtypeStruct((_N, 128), jnp.float32),
  )(hlo, hhi, agg, agg, deg, w2, b2)


def kernel(x, c, edge_index, W1_0, b1_0, Wc_0, bc_0, W2_0, b2_0,
           W1_1, b1_1, Wc_1, bc_1, W2_1, b2_1):
  src = edge_index[0]
  dst = edge_index[1]
  pad = _EPAD - _E
  srcr = jnp.concatenate([src, jnp.zeros((pad,), jnp.int32)]) \
      .reshape(_NROWS, _CHUNK)
  dstr = jnp.concatenate([dst, jnp.full((pad,), _N, jnp.int32)]) \
      .reshape(_NROWS, _CHUNK)

  hlo, hhi, c1 = _tc_pre(x, c, W1_0[:128], W1_0[128:],
                         b1_0.reshape(1, _H), Wc_0, bc_0.reshape(1, _H))
  agg0, deg = _sc_agg_deg(hlo, hhi, srcr, dstr)
  deg2 = deg.reshape(_N, 1)
  h1lo, h1hi = _tc_mid(hlo, hhi, agg0, deg2, c1, W2_0, b2_0.reshape(1, _H),
                       W1_1[:_H], W1_1[_H:], b1_1.reshape(1, _H))
  agg1, _unused_deg = _sc_agg_deg(h1lo, h1hi, srcr, dstr)
  return _tc_fin(h1lo, h1hi, agg1, deg2, W2_1, b2_1.reshape(1, 128))


# speedup vs baseline: 1.0733x; 1.0733x over previous
import functools

import jax
import jax.numpy as jnp
from jax import lax
from jax.experimental import pallas as pl
from jax.experimental.pallas import tpu as pltpu
from jax.experimental.pallas import tpu_sc as plsc

_N = 50000
_E = 800000
_H = 64
_HH = 32
_CHUNK = 128
_TILES = 16
_CPT = 392
_EPAD = _TILES * _CPT * _CHUNK
_NROWS = _EPAD // _CHUNK
_NBUF = 4
_NP = 51200
_ZB = _NP // (_TILES * _CHUNK)
_PK = 12544
_PKT = _PK // _TILES
_R = 2048


def _sc_aggregate(with_deg):
  mesh = plsc.VectorSubcoreMesh(core_axis_name="c", subcore_axis_name="s")
  out_type = [jax.ShapeDtypeStruct((2, 4 * _PK, _HH), jnp.float32)]
  scratch = [
      pltpu.VMEM((8, _CHUNK), jnp.int32),
      pltpu.VMEM((8, _CHUNK), jnp.int32),
      pltpu.VMEM((_NBUF, _CHUNK, _HH), jnp.float32),
      pltpu.VMEM_SHARED((_NP, _HH), jnp.float32),
      pltpu.SemaphoreType.DMA,
      pltpu.SemaphoreType.DMA,
      pltpu.SemaphoreType.DMA,
  ]
  if with_deg:
    out_type.append(jax.ShapeDtypeStruct((_N,), jnp.float32))
    scratch += [
        pltpu.VMEM((_CHUNK,), jnp.float32),
        pltpu.VMEM((3200,), jnp.float32),
        pltpu.VMEM_SHARED((_NP,), jnp.float32),
        pltpu.SemaphoreType.DMA,
    ]

  def body(h_lo, h_hi, srcr, dstr, agg_out, *rest):
    if with_deg:
      (deg_out, src_v, dst_v, rows_v, agg_sh, sem_i, sem_g, sem_s,
       ones_v, dzero_v, deg_sh, sem_d) = rest
    else:
      src_v, dst_v, rows_v, agg_sh, sem_i, sem_g, sem_s = rest
    c = lax.axis_index("c")
    s = lax.axis_index("s")

    def _zrow(i, _):
      rows_v[0, i, pl.ds(0, 16)] = jnp.zeros((16,), jnp.float32)
      rows_v[0, i, pl.ds(16, 16)] = jnp.zeros((16,), jnp.float32)
      return 0
    lax.fori_loop(0, _CHUNK, _zrow, 0)

    def _zshared(j, _):
      pltpu.sync_copy(rows_v.at[0],
                      agg_sh.at[pl.ds((s * _ZB + j) * _CHUNK, _CHUNK)])
      return 0
    lax.fori_loop(0, _ZB, _zshared, 0)

    if with_deg:
      def _zd(i, _):
        dzero_v[pl.ds(i * 16, 16)] = jnp.zeros((16,), jnp.float32)
        return 0
      lax.fori_loop(0, 200, _zd, 0)
      pltpu.sync_copy(dzero_v, deg_sh.at[pl.ds(s * 3200, 3200)])

      def _ones(i, _):
        ones_v[pl.ds(i * 16, 16)] = jnp.ones((16,), jnp.float32)
        return 0
      lax.fori_loop(0, _CHUNK // 16, _ones, 0)

    plsc.subcore_barrier()

    row0 = s * _CPT

    def issue_idx(j, b):
      pltpu.async_copy(srcr.at[row0 + j], src_v.at[b], sem_i)
      pltpu.async_copy(dstr.at[row0 + j], dst_v.at[b], sem_i)

    def issue_gather(ib, rb):
      @pl.when(c == 0)
      def _():
        pltpu.async_copy(h_lo.at[src_v.at[ib]], rows_v.at[rb], sem_g)

      @pl.when(c == 1)
      def _():
        pltpu.async_copy(h_hi.at[src_v.at[ib]], rows_v.at[rb], sem_g)

    def wait_idx_pair():
      pltpu.make_async_copy(dstr.at[0], src_v.at[0], sem_i).wait()
      pltpu.make_async_copy(dstr.at[0], dst_v.at[0], sem_i).wait()

    def wait_chunk(sem):
      pltpu.make_async_copy(h_lo.at[pl.ds(0, _CHUNK)], rows_v.at[0],
                            sem).wait()

    if with_deg:
      def wait_deg():
        pltpu.make_async_copy(ones_v, deg_sh.at[pl.ds(0, _CHUNK)],
                              sem_d).wait()

    for b in range(4):
      issue_idx(b, b)
    wait_idx_pair()
    wait_idx_pair()
    issue_gather(0, 0)
    issue_gather(1, 1)

    def grp(g, _):
      for b in range(8):
        j = g * 8 + b

        @pl.when(j + 4 < _CPT)
        def _():
          issue_idx(j + 4, (b + 4) % 8)

        @pl.when(j >= 2)
        def _():
          wait_chunk(sem_s)

        @pl.when(j + 2 < _CPT)
        def _():
          wait_idx_pair()
          issue_gather((b + 2) % 8, (b + 2) % _NBUF)

        wait_chunk(sem_g)
        pltpu.async_copy(rows_v.at[b % _NBUF], agg_sh.at[dst_v.at[b % 8]],
                         sem_s, add=True)

        if with_deg:
          @pl.when(c == 0)
          def _():
            @pl.when(j >= 2)
            def _():
              wait_deg()
            pltpu.async_copy(ones_v, deg_sh.at[dst_v.at[b % 8]], sem_d,
                             add=True)
      return 0
    lax.fori_loop(0, _CPT // 8, grp, 0)

    wait_chunk(sem_s)
    wait_chunk(sem_s)
    if with_deg:
      @pl.when(c == 0)
      def _():
        wait_deg()
        wait_deg()

    plsc.subcore_barrier()

    pltpu.sync_copy(agg_sh.at[pl.ds(s * 4 * _PKT, 4 * _PKT)],
                    agg_out.at[c, pl.ds(s * 4 * _PKT, 4 * _PKT)])

    if with_deg:
      @pl.when(jnp.logical_and(c == 0, s == 0))
      def _():
        pltpu.sync_copy(deg_sh.at[pl.ds(0, _N)], deg_out)

  return pl.kernel(
      body, out_type=out_type, mesh=mesh, scratch_types=scratch,
      compiler_params=pltpu.CompilerParams(use_tc_tiling_on_sc=False))


_sc_agg_deg = _sc_aggregate(True)
_sc_agg = _sc_aggregate(False)


def _full(shape):
  return pl.BlockSpec(shape, lambda i: tuple(0 for _ in shape))


def _tc_pre(x, cc, w1x, w1c, b1, wc, bc):
  def body(x_r, c_r, w1x_r, w1c_r, b1_r, wc_r, bc_r, lo_r, hi_r, c1_r):
    h = jnp.maximum(
        jnp.dot(x_r[...], w1x_r[...], preferred_element_type=jnp.float32)
        + jnp.dot(c_r[...], w1c_r[...], preferred_element_type=jnp.float32)
        + b1_r[...], 0.0)
    c1 = jnp.maximum(
        jnp.dot(c_r[...], wc_r[...], preferred_element_type=jnp.float32)
        + bc_r[...], 0.0)
    lo_r[...] = h[:, :_HH]
    hi_r[...] = h[:, _HH:]
    c1_r[...] = c1

  hh = pl.BlockSpec((_R, _HH), lambda i: (i, 0))
  return pl.pallas_call(
      body,
      grid=((_N + _R - 1) // _R,),
      in_specs=[
          pl.BlockSpec((_R, 128), lambda i: (i, 0)),
          pl.BlockSpec((_R, 16), lambda i: (i, 0)),
          _full((128, _H)), _full((16, _H)), _full((1, _H)),
          _full((16, _H)), _full((1, _H)),
      ],
      out_specs=[hh, hh, pl.BlockSpec((_R, _H), lambda i: (i, 0))],
      out_shape=[
          jax.ShapeDtypeStruct((_N, _HH), jnp.float32),
          jax.ShapeDtypeStruct((_N, _HH), jnp.float32),
          jax.ShapeDtypeStruct((_N, _H), jnp.float32),
      ],
  )(x, cc, w1x, w1c, b1, wc, bc)


def _tc_mid(hlo, hhi, agg, deg, c1, w2, b2, w1a, w1b, b11):
  def body(hlo_r, hhi_r, alo_r, ahi_r, deg_r, c1_r, w2_r, b2_r, w1a_r, w1b_r,
           b11_r, lo_r, hi_r):
    inv = 1.0 / jnp.maximum(deg_r[...], 1.0)
    u = jnp.concatenate(
        [hlo_r[...] + alo_r[0] * inv, hhi_r[...] + ahi_r[0] * inv], axis=1)
    x1 = jnp.dot(u, w2_r[...], preferred_element_type=jnp.float32) + b2_r[...]
    h1 = jnp.maximum(
        jnp.dot(x1, w1a_r[...], preferred_element_type=jnp.float32)
        + jnp.dot(c1_r[...], w1b_r[...], preferred_element_type=jnp.float32)
        + b11_r[...], 0.0)
    lo_r[...] = h1[:, :_HH]
    hi_r[...] = h1[:, _HH:]

  hh = pl.BlockSpec((_R, _HH), lambda i: (i, 0))
  alo = pl.BlockSpec((1, _R, _HH), lambda i: (0, i, 0))
  ahi = pl.BlockSpec((1, _R, _HH), lambda i: (1, i, 0))
  return pl.pallas_call(
      body,
      grid=((_N + _R - 1) // _R,),
      in_specs=[
          hh, hh, alo, ahi,
          pl.BlockSpec((_R, 1), lambda i: (i, 0)),
          pl.BlockSpec((_R, _H), lambda i: (i, 0)),
          _full((_H, _H)), _full((1, _H)),
          _full((_H, _H)), _full((_H, _H)), _full((1, _H)),
      ],
      out_specs=[hh, hh],
      out_shape=[
          jax.ShapeDtypeStruct((_N, _HH), jnp.float32),
          jax.ShapeDtypeStruct((_N, _HH), jnp.float32),
      ],
  )(hlo, hhi, agg, agg, deg, c1, w2, b2, w1a, w1b, b11)


def _tc_fin(hlo, hhi, agg, deg, w2, b2):
  def body(hlo_r, hhi_r, alo_r, ahi_r, deg_r, w2_r, b2_r, out_r):
    inv = 1.0 / jnp.maximum(deg_r[...], 1.0)
    u = jnp.concatenate(
        [hlo_r[...] + alo_r[0] * inv, hhi_r[...] + ahi_r[0] * inv], axis=1)
    out_r[...] = (
        jnp.dot(u, w2_r[...], preferred_element_type=jnp.float32) + b2_r[...])

  hh = pl.BlockSpec((_R, _HH), lambda i: (i, 0))
  alo = pl.BlockSpec((1, _R, _HH), lambda i: (0, i, 0))
  ahi = pl.BlockSpec((1, _R, _HH), lambda i: (1, i, 0))
  return pl.pallas_call(
      body,
      grid=((_N + _R - 1) // _R,),
      in_specs=[
          hh, hh, alo, ahi,
          pl.BlockSpec((_R, 1), lambda i: (i, 0)),
          _full((_H, 128)), _full((1, 128)),
      ],
      out_specs=pl.BlockSpec((_R, 128), lambda i: (i, 0)),
      out_shape=jax.ShapeDtypeStruct((_N, 128), jnp.float32),
  )(hlo, hhi, agg, agg, deg, w2, b2)


def kernel(x, c, edge_index, W1_0, b1_0, Wc_0, bc_0, W2_0, b2_0,
           W1_1, b1_1, Wc_1, bc_1, W2_1, b2_1):
  src = edge_index[0]
  dst = edge_index[1]
  pad = _EPAD - _E
  srcr = jnp.concatenate([src, jnp.zeros((pad,), jnp.int32)]) \
      .reshape(_NROWS, _CHUNK)
  dstr = jnp.concatenate([dst, jnp.full((pad,), _N, jnp.int32)]) \
      .reshape(_NROWS, _CHUNK)

  hlo, hhi, c1 = _tc_pre(x, c, W1_0[:128], W1_0[128:],
                         b1_0.reshape(1, _H), Wc_0, bc_0.reshape(1, _H))
  agg0, deg = _sc_agg_deg(hlo, hhi, srcr, dstr)
  deg2 = deg.reshape(_N, 1)
  h1lo, h1hi = _tc_mid(hlo, hhi, agg0, deg2, c1, W2_0, b2_0.reshape(1, _H),
                       W1_1[:_H], W1_1[_H:], b1_1.reshape(1, _H))
  agg1 = _sc_agg(h1lo, h1hi, srcr, dstr)
  if isinstance(agg1, (list, tuple)):
    agg1 = agg1[0]
  return _tc_fin(h1lo, h1hi, agg1, deg2, W2_1, b2_1.reshape(1, 128))


# speedup vs baseline: 1.2064x; 1.1240x over previous
import functools

import jax
import jax.numpy as jnp
from jax import lax
from jax.experimental import pallas as pl
from jax.experimental.pallas import tpu as pltpu
from jax.experimental.pallas import tpu_sc as plsc

_N = 50000
_E = 800000
_H = 64
_HH = 32
_CHUNK = 128
_TILES = 16
_NCH = _E // _CHUNK
_CPT_A = 391
_CPT_B = 390
_NGRP = 50
_NBUF = 4
_NP = 51200
_ZB = _NP // (_TILES * _CHUNK)
_PK = 12544
_PKT = _PK // _TILES
_R = 2048


def _sc_aggregate(with_deg):
  mesh = plsc.VectorSubcoreMesh(core_axis_name="c", subcore_axis_name="s")
  out_type = [jax.ShapeDtypeStruct((2, 4 * _PK, _HH), jnp.float32)]
  scratch = [
      pltpu.VMEM((8, _CHUNK), jnp.int32),
      pltpu.VMEM((8, _CHUNK), jnp.int32),
      pltpu.VMEM((_NBUF, _CHUNK, _HH), jnp.float32),
      pltpu.VMEM_SHARED((_NP, _HH), jnp.float32),
      pltpu.SemaphoreType.DMA,
      pltpu.SemaphoreType.DMA,
      pltpu.SemaphoreType.DMA,
  ]
  if with_deg:
    out_type.append(jax.ShapeDtypeStruct((_N,), jnp.float32))
    scratch += [
        pltpu.VMEM((_CHUNK,), jnp.float32),
        pltpu.VMEM((3200,), jnp.float32),
        pltpu.VMEM_SHARED((_NP,), jnp.float32),
        pltpu.SemaphoreType.DMA,
    ]

  def body(h_lo, h_hi, ei, agg_out, *rest):
    if with_deg:
      (deg_out, src_v, dst_v, rows_v, agg_sh, sem_i, sem_g, sem_s,
       ones_v, dzero_v, deg_sh, sem_d) = rest
    else:
      src_v, dst_v, rows_v, agg_sh, sem_i, sem_g, sem_s = rest
    c = lax.axis_index("c")
    s = lax.axis_index("s")

    def _zrow(i, _):
      rows_v[0, i, pl.ds(0, 16)] = jnp.zeros((16,), jnp.float32)
      rows_v[0, i, pl.ds(16, 16)] = jnp.zeros((16,), jnp.float32)
      return 0
    lax.fori_loop(0, _CHUNK, _zrow, 0)

    def _zshared(j, _):
      pltpu.sync_copy(rows_v.at[0],
                      agg_sh.at[pl.ds((s * _ZB + j) * _CHUNK, _CHUNK)])
      return 0
    lax.fori_loop(0, _ZB, _zshared, 0)

    if with_deg:
      def _zd(i, _):
        dzero_v[pl.ds(i * 16, 16)] = jnp.zeros((16,), jnp.float32)
        return 0
      lax.fori_loop(0, 200, _zd, 0)
      pltpu.sync_copy(dzero_v, deg_sh.at[pl.ds(s * 3200, 3200)])

      def _ones(i, _):
        ones_v[pl.ds(i * 16, 16)] = jnp.ones((16,), jnp.float32)
        return 0
      lax.fori_loop(0, _CHUNK // 16, _ones, 0)

    plsc.subcore_barrier()

    cpt = jnp.where(s < 10, _CPT_A, _CPT_B)
    row0 = jnp.where(s < 10, s * _CPT_A, 10 * _CPT_A + (s - 10) * _CPT_B)

    def issue_idx(j, b):
      e0 = (row0 + j) * _CHUNK
      pltpu.async_copy(ei.at[0, pl.ds(e0, _CHUNK)], src_v.at[b], sem_i)
      pltpu.async_copy(ei.at[1, pl.ds(e0, _CHUNK)], dst_v.at[b], sem_i)

    def issue_gather(ib, rb):
      @pl.when(c == 0)
      def _():
        pltpu.async_copy(h_lo.at[src_v.at[ib]], rows_v.at[rb], sem_g)

      @pl.when(c == 1)
      def _():
        pltpu.async_copy(h_hi.at[src_v.at[ib]], rows_v.at[rb], sem_g)

    def wait_idx_pair():
      pltpu.make_async_copy(ei.at[0, pl.ds(0, _CHUNK)], src_v.at[0],
                            sem_i).wait()
      pltpu.make_async_copy(ei.at[0, pl.ds(0, _CHUNK)], dst_v.at[0],
                            sem_i).wait()

    def wait_chunk(sem):
      pltpu.make_async_copy(h_lo.at[pl.ds(0, _CHUNK)], rows_v.at[0],
                            sem).wait()

    if with_deg:
      def wait_deg():
        pltpu.make_async_copy(ones_v, deg_sh.at[pl.ds(0, _CHUNK)],
                              sem_d).wait()

    for b in range(4):
      issue_idx(b, b)
    wait_idx_pair()
    wait_idx_pair()
    issue_gather(0, 0)
    issue_gather(1, 1)

    def grp(g, _):
      for b in range(8):
        j = g * 8 + b

        @pl.when(j + 4 < cpt)
        def _():
          issue_idx(j + 4, (b + 4) % 8)

        @pl.when(jnp.logical_and(j >= 2, j - 2 < cpt))
        def _():
          wait_chunk(sem_s)

        @pl.when(j + 2 < cpt)
        def _():
          wait_idx_pair()
          issue_gather((b + 2) % 8, (b + 2) % _NBUF)

        @pl.when(j < cpt)
        def _():
          wait_chunk(sem_g)
          pltpu.async_copy(rows_v.at[b % _NBUF], agg_sh.at[dst_v.at[b % 8]],
                           sem_s, add=True)

        if with_deg:
          @pl.when(c == 0)
          def _():
            @pl.when(jnp.logical_and(j >= 2, j - 2 < cpt))
            def _():
              wait_deg()

            @pl.when(j < cpt)
            def _():
              pltpu.async_copy(ones_v, deg_sh.at[dst_v.at[b % 8]], sem_d,
                               add=True)
      return 0
    lax.fori_loop(0, _NGRP, grp, 0)

    plsc.subcore_barrier()

    pltpu.sync_copy(agg_sh.at[pl.ds(s * 4 * _PKT, 4 * _PKT)],
                    agg_out.at[c, pl.ds(s * 4 * _PKT, 4 * _PKT)])

    if with_deg:
      @pl.when(jnp.logical_and(c == 0, s == 0))
      def _():
        pltpu.sync_copy(deg_sh.at[pl.ds(0, _N)], deg_out)

  return pl.kernel(
      body, out_type=out_type, mesh=mesh, scratch_types=scratch,
      compiler_params=pltpu.CompilerParams(use_tc_tiling_on_sc=False))


_sc_agg_deg = _sc_aggregate(True)
_sc_agg = _sc_aggregate(False)


def _full(shape):
  return pl.BlockSpec(shape, lambda i: tuple(0 for _ in shape))


def _tc_pre(x, cc, w1x, w1c, b1, wc, bc):
  def body(x_r, c_r, w1x_r, w1c_r, b1_r, wc_r, bc_r, lo_r, hi_r, c1_r):
    h = jnp.maximum(
        jnp.dot(x_r[...], w1x_r[...], preferred_element_type=jnp.float32)
        + jnp.dot(c_r[...], w1c_r[...], preferred_element_type=jnp.float32)
        + b1_r[...], 0.0)
    c1 = jnp.maximum(
        jnp.dot(c_r[...], wc_r[...], preferred_element_type=jnp.float32)
        + bc_r[...], 0.0)
    lo_r[...] = h[:, :_HH]
    hi_r[...] = h[:, _HH:]
    c1_r[...] = c1

  hh = pl.BlockSpec((_R, _HH), lambda i: (i, 0))
  return pl.pallas_call(
      body,
      grid=((_N + _R - 1) // _R,),
      in_specs=[
          pl.BlockSpec((_R, 128), lambda i: (i, 0)),
          pl.BlockSpec((_R, 16), lambda i: (i, 0)),
          _full((128, _H)), _full((16, _H)), _full((1, _H)),
          _full((16, _H)), _full((1, _H)),
      ],
      out_specs=[hh, hh, pl.BlockSpec((_R, _H), lambda i: (i, 0))],
      out_shape=[
          jax.ShapeDtypeStruct((_N, _HH), jnp.float32),
          jax.ShapeDtypeStruct((_N, _HH), jnp.float32),
          jax.ShapeDtypeStruct((_N, _H), jnp.float32),
      ],
  )(x, cc, w1x, w1c, b1, wc, bc)


def _tc_mid(hlo, hhi, agg, deg, c1, w2, b2, w1a, w1b, b11):
  def body(hlo_r, hhi_r, alo_r, ahi_r, deg_r, c1_r, w2_r, b2_r, w1a_r, w1b_r,
           b11_r, lo_r, hi_r):
    inv = 1.0 / jnp.maximum(deg_r[...], 1.0)
    u = jnp.concatenate(
        [hlo_r[...] + alo_r[0] * inv, hhi_r[...] + ahi_r[0] * inv], axis=1)
    x1 = jnp.dot(u, w2_r[...], preferred_element_type=jnp.float32) + b2_r[...]
    h1 = jnp.maximum(
        jnp.dot(x1, w1a_r[...], preferred_element_type=jnp.float32)
        + jnp.dot(c1_r[...], w1b_r[...], preferred_element_type=jnp.float32)
        + b11_r[...], 0.0)
    lo_r[...] = h1[:, :_HH]
    hi_r[...] = h1[:, _HH:]

  hh = pl.BlockSpec((_R, _HH), lambda i: (i, 0))
  alo = pl.BlockSpec((1, _R, _HH), lambda i: (0, i, 0))
  ahi = pl.BlockSpec((1, _R, _HH), lambda i: (1, i, 0))
  return pl.pallas_call(
      body,
      grid=((_N + _R - 1) // _R,),
      in_specs=[
          hh, hh, alo, ahi,
          pl.BlockSpec((_R, 1), lambda i: (i, 0)),
          pl.BlockSpec((_R, _H), lambda i: (i, 0)),
          _full((_H, _H)), _full((1, _H)),
          _full((_H, _H)), _full((_H, _H)), _full((1, _H)),
      ],
      out_specs=[hh, hh],
      out_shape=[
          jax.ShapeDtypeStruct((_N, _HH), jnp.float32),
          jax.ShapeDtypeStruct((_N, _HH), jnp.float32),
      ],
  )(hlo, hhi, agg, agg, deg, c1, w2, b2, w1a, w1b, b11)


def _tc_fin(hlo, hhi, agg, deg, w2, b2):
  def body(hlo_r, hhi_r, alo_r, ahi_r, deg_r, w2_r, b2_r, out_r):
    inv = 1.0 / jnp.maximum(deg_r[...], 1.0)
    u = jnp.concatenate(
        [hlo_r[...] + alo_r[0] * inv, hhi_r[...] + ahi_r[0] * inv], axis=1)
    out_r[...] = (
        jnp.dot(u, w2_r[...], preferred_element_type=jnp.float32) + b2_r[...])

  hh = pl.BlockSpec((_R, _HH), lambda i: (i, 0))
  alo = pl.BlockSpec((1, _R, _HH), lambda i: (0, i, 0))
  ahi = pl.BlockSpec((1, _R, _HH), lambda i: (1, i, 0))
  return pl.pallas_call(
      body,
      grid=((_N + _R - 1) // _R,),
      in_specs=[
          hh, hh, alo, ahi,
          pl.BlockSpec((_R, 1), lambda i: (i, 0)),
          _full((_H, 128)), _full((1, 128)),
      ],
      out_specs=pl.BlockSpec((_R, 128), lambda i: (i, 0)),
      out_shape=jax.ShapeDtypeStruct((_N, 128), jnp.float32),
  )(hlo, hhi, agg, agg, deg, w2, b2)


def kernel(x, c, edge_index, W1_0, b1_0, Wc_0, bc_0, W2_0, b2_0,
           W1_1, b1_1, Wc_1, bc_1, W2_1, b2_1):
  hlo, hhi, c1 = _tc_pre(x, c, W1_0[:128], W1_0[128:],
                         b1_0.reshape(1, _H), Wc_0, bc_0.reshape(1, _H))
  agg0, deg = _sc_agg_deg(hlo, hhi, edge_index)
  deg2 = deg.reshape(_N, 1)
  h1lo, h1hi = _tc_mid(hlo, hhi, agg0, deg2, c1, W2_0, b2_0.reshape(1, _H),
                       W1_1[:_H], W1_1[_H:], b1_1.reshape(1, _H))
  agg1 = _sc_agg(h1lo, h1hi, edge_index)
  if isinstance(agg1, (list, tuple)):
    agg1 = agg1[0]
  return _tc_fin(h1lo, h1hi, agg1, deg2, W2_1, b2_1.reshape(1, 128))


# speedup vs baseline: 1.5301x; 1.2683x over previous
import functools

import jax
import jax.numpy as jnp
from jax import lax
from jax.experimental import pallas as pl
from jax.experimental.pallas import tpu as pltpu
from jax.experimental.pallas import tpu_sc as plsc

_N = 50000
_E = 800000
_H = 64
_HH = 32
_CHUNK = 128
_TILES = 16
_NCH = _E // _CHUNK
_CPT_A = 391
_CPT_B = 390
_NGRP = 50
_NBUF = 4
_NP = 51200
_ZB = _NP // (_TILES * _CHUNK)
_PK = 12544
_PKT = _PK // _TILES
_N4 = _N // 4
_PB = 512
_GRID = (_N4 + _PB - 1) // _PB


def _sc_aggregate(with_deg):
  mesh = plsc.VectorSubcoreMesh(core_axis_name="c", subcore_axis_name="s")
  out_type = [jax.ShapeDtypeStruct((2, 4 * _PK, _HH), jnp.float32)]
  scratch = [
      pltpu.VMEM((8, _CHUNK), jnp.int32),
      pltpu.VMEM((8, _CHUNK), jnp.int32),
      pltpu.VMEM((_NBUF, _CHUNK, _HH), jnp.float32),
      pltpu.VMEM_SHARED((_NP, _HH), jnp.float32),
      pltpu.SemaphoreType.DMA,
      pltpu.SemaphoreType.DMA,
      pltpu.SemaphoreType.DMA,
  ]
  if with_deg:
    out_type.append(jax.ShapeDtypeStruct((_N,), jnp.float32))
    scratch += [
        pltpu.VMEM((_CHUNK,), jnp.float32),
        pltpu.VMEM((3200,), jnp.float32),
        pltpu.VMEM_SHARED((_NP,), jnp.float32),
        pltpu.SemaphoreType.DMA,
    ]

  def body(h_lo, h_hi, ei, agg_out, *rest):
    if with_deg:
      (deg_out, src_v, dst_v, rows_v, agg_sh, sem_i, sem_g, sem_s,
       ones_v, dzero_v, deg_sh, sem_d) = rest
    else:
      src_v, dst_v, rows_v, agg_sh, sem_i, sem_g, sem_s = rest
    c = lax.axis_index("c")
    s = lax.axis_index("s")

    def _zrow(i, _):
      rows_v[0, i, pl.ds(0, 16)] = jnp.zeros((16,), jnp.float32)
      rows_v[0, i, pl.ds(16, 16)] = jnp.zeros((16,), jnp.float32)
      return 0
    lax.fori_loop(0, _CHUNK, _zrow, 0)

    def _zshared(j, _):
      pltpu.sync_copy(rows_v.at[0],
                      agg_sh.at[pl.ds((s * _ZB + j) * _CHUNK, _CHUNK)])
      return 0
    lax.fori_loop(0, _ZB, _zshared, 0)

    if with_deg:
      def _zd(i, _):
        dzero_v[pl.ds(i * 16, 16)] = jnp.zeros((16,), jnp.float32)
        return 0
      lax.fori_loop(0, 200, _zd, 0)
      pltpu.sync_copy(dzero_v, deg_sh.at[pl.ds(s * 3200, 3200)])

      def _ones(i, _):
        ones_v[pl.ds(i * 16, 16)] = jnp.ones((16,), jnp.float32)
        return 0
      lax.fori_loop(0, _CHUNK // 16, _ones, 0)

    plsc.subcore_barrier()

    cpt = jnp.where(s < 10, _CPT_A, _CPT_B)
    row0 = jnp.where(s < 10, s * _CPT_A, 10 * _CPT_A + (s - 10) * _CPT_B)

    def issue_idx(j, b):
      e0 = (row0 + j) * _CHUNK
      pltpu.async_copy(ei.at[0, pl.ds(e0, _CHUNK)], src_v.at[b], sem_i)
      pltpu.async_copy(ei.at[1, pl.ds(e0, _CHUNK)], dst_v.at[b], sem_i)

    def issue_gather(ib, rb):
      @pl.when(c == 0)
      def _():
        pltpu.async_copy(h_lo.at[src_v.at[ib]], rows_v.at[rb], sem_g)

      @pl.when(c == 1)
      def _():
        pltpu.async_copy(h_hi.at[src_v.at[ib]], rows_v.at[rb], sem_g)

    def wait_idx_pair():
      pltpu.make_async_copy(ei.at[0, pl.ds(0, _CHUNK)], src_v.at[0],
                            sem_i).wait()
      pltpu.make_async_copy(ei.at[0, pl.ds(0, _CHUNK)], dst_v.at[0],
                            sem_i).wait()

    def wait_chunk(sem):
      pltpu.make_async_copy(h_lo.at[pl.ds(0, _CHUNK)], rows_v.at[0],
                            sem).wait()

    if with_deg:
      def wait_deg():
        pltpu.make_async_copy(ones_v, deg_sh.at[pl.ds(0, _CHUNK)],
                              sem_d).wait()

    for b in range(4):
      issue_idx(b, b)
    wait_idx_pair()
    wait_idx_pair()
    issue_gather(0, 0)
    issue_gather(1, 1)

    def grp(g, _):
      for b in range(8):
        j = g * 8 + b

        @pl.when(j + 4 < cpt)
        def _():
          issue_idx(j + 4, (b + 4) % 8)

        @pl.when(jnp.logical_and(j >= 2, j - 2 < cpt))
        def _():
          wait_chunk(sem_s)

        @pl.when(j + 2 < cpt)
        def _():
          wait_idx_pair()
          issue_gather((b + 2) % 8, (b + 2) % _NBUF)

        @pl.when(j < cpt)
        def _():
          wait_chunk(sem_g)
          pltpu.async_copy(rows_v.at[b % _NBUF], agg_sh.at[dst_v.at[b % 8]],
                           sem_s, add=True)

        if with_deg:
          @pl.when(c == 0)
          def _():
            @pl.when(jnp.logical_and(j >= 2, j - 2 < cpt))
            def _():
              wait_deg()

            @pl.when(j < cpt)
            def _():
              pltpu.async_copy(ones_v, deg_sh.at[dst_v.at[b % 8]], sem_d,
                               add=True)
      return 0
    lax.fori_loop(0, _NGRP, grp, 0)

    plsc.subcore_barrier()

    pltpu.sync_copy(agg_sh.at[pl.ds(s * 4 * _PKT, 4 * _PKT)],
                    agg_out.at[c, pl.ds(s * 4 * _PKT, 4 * _PKT)])

    if with_deg:
      @pl.when(jnp.logical_and(c == 0, s == 0))
      def _():
        pltpu.sync_copy(deg_sh.at[pl.ds(0, _N)], deg_out)

  return pl.kernel(
      body, out_type=out_type, mesh=mesh, scratch_types=scratch,
      compiler_params=pltpu.CompilerParams(use_tc_tiling_on_sc=False))


_sc_agg_deg = _sc_aggregate(True)
_sc_agg = _sc_aggregate(False)


def _full(shape):
  return pl.BlockSpec(shape, lambda i: tuple(0 for _ in shape))


def _tc_pre(xp, cp, wxlo, wxhi, wclo, wchi, blo, bhi, wcbd, bcp):
  def body(x_r, c_r, wxlo_r, wxhi_r, wclo_r, wchi_r, blo_r, bhi_r, wcbd_r,
           bcp_r, lo_r, hi_r, c1_r):
    lo_r[...] = jnp.maximum(
        jnp.dot(x_r[...], wxlo_r[...], preferred_element_type=jnp.float32)
        + jnp.dot(c_r[...], wclo_r[...], preferred_element_type=jnp.float32)
        + blo_r[...], 0.0)
    hi_r[...] = jnp.maximum(
        jnp.dot(x_r[...], wxhi_r[...], preferred_element_type=jnp.float32)
        + jnp.dot(c_r[...], wchi_r[...], preferred_element_type=jnp.float32)
        + bhi_r[...], 0.0)
    c1_r[...] = jnp.maximum(
        jnp.dot(c_r[...], wcbd_r[...], preferred_element_type=jnp.float32)
        + bcp_r[...], 0.0)

  pk = pl.BlockSpec((_PB, 128), lambda i: (i, 0))
  return pl.pallas_call(
      body,
      grid=(_GRID,),
      in_specs=[
          pl.BlockSpec((_PB, 512), lambda i: (i, 0)),
          pl.BlockSpec((_PB, 64), lambda i: (i, 0)),
          _full((512, 128)), _full((512, 128)),
          _full((64, 128)), _full((64, 128)),
          _full((1, 128)), _full((1, 128)),
          _full((64, 256)), _full((1, 256)),
      ],
      out_specs=[pk, pk, pl.BlockSpec((_PB, 256), lambda i: (i, 0))],
      out_shape=[
          jax.ShapeDtypeStruct((_N4, 128), jnp.float32),
          jax.ShapeDtypeStruct((_N4, 128), jnp.float32),
          jax.ShapeDtypeStruct((_N4, 256), jnp.float32),
      ],
  )(xp, cp, wxlo, wxhi, wclo, wchi, blo, bhi, wcbd, bcp)


def _tc_mid(hlo, hhi, agg, deg4, c1p, w2lo, w2hi, b2p, w1abd, w1bbd, b11p,
            sello, selhi, s32):
  def body(hlo_r, hhi_r, alo_r, ahi_r, deg_r, c1_r, w2lo_r, w2hi_r, b2_r,
           w1a_r, w1b_r, b11_r, sello_r, selhi_r, s32_r, lo_r, hi_r):
    inv4 = 1.0 / jnp.maximum(deg_r[...], 1.0)
    invb = jnp.dot(inv4, s32_r[...], preferred_element_type=jnp.float32)
    plo = hlo_r[...] + alo_r[0] * invb
    phi = hhi_r[...] + ahi_r[0] * invb
    x1 = (jnp.dot(plo, w2lo_r[...], preferred_element_type=jnp.float32)
          + jnp.dot(phi, w2hi_r[...], preferred_element_type=jnp.float32)
          + b2_r[...])
    h1 = jnp.maximum(
        jnp.dot(x1, w1a_r[...], preferred_element_type=jnp.float32)
        + jnp.dot(c1_r[...], w1b_r[...], preferred_element_type=jnp.float32)
        + b11_r[...], 0.0)
    lo_r[...] = jnp.dot(h1, sello_r[...], preferred_element_type=jnp.float32)
    hi_r[...] = jnp.dot(h1, selhi_r[...], preferred_element_type=jnp.float32)

  pk = pl.BlockSpec((_PB, 128), lambda i: (i, 0))
  alo = pl.BlockSpec((1, _PB, 128), lambda i: (0, i, 0))
  ahi = pl.BlockSpec((1, _PB, 128), lambda i: (1, i, 0))
  return pl.pallas_call(
      body,
      grid=(_GRID,),
      in_specs=[
          pk, pk, alo, ahi,
          pl.BlockSpec((_PB, 4), lambda i: (i, 0)),
          pl.BlockSpec((_PB, 256), lambda i: (i, 0)),
          _full((128, 256)), _full((128, 256)), _full((1, 256)),
          _full((256, 256)), _full((256, 256)), _full((1, 256)),
          _full((256, 128)), _full((256, 128)), _full((4, 128)),
      ],
      out_specs=[pk, pk],
      out_shape=[
          jax.ShapeDtypeStruct((_N4, 128), jnp.float32),
          jax.ShapeDtypeStruct((_N4, 128), jnp.float32),
      ],
  )(hlo, hhi, agg, agg, deg4, c1p, w2lo, w2hi, b2p, w1abd, w1bbd, b11p,
    sello, selhi, s32)


def _tc_fin(hlo, hhi, agg, deg4, w2lo, w2hi, b2p, s32):
  def body(hlo_r, hhi_r, alo_r, ahi_r, deg_r, w2lo_r, w2hi_r, b2_r, s32_r,
           out_r):
    inv4 = 1.0 / jnp.maximum(deg_r[...], 1.0)
    invb = jnp.dot(inv4, s32_r[...], preferred_element_type=jnp.float32)
    plo = hlo_r[...] + alo_r[0] * invb
    phi = hhi_r[...] + ahi_r[0] * invb
    out_r[...] = (
        jnp.dot(plo, w2lo_r[...], preferred_element_type=jnp.float32)
        + jnp.dot(phi, w2hi_r[...], preferred_element_type=jnp.float32)
        + b2_r[...])

  pk = pl.BlockSpec((_PB, 128), lambda i: (i, 0))
  alo = pl.BlockSpec((1, _PB, 128), lambda i: (0, i, 0))
  ahi = pl.BlockSpec((1, _PB, 128), lambda i: (1, i, 0))
  return pl.pallas_call(
      body,
      grid=(_GRID,),
      in_specs=[
          pk, pk, alo, ahi,
          pl.BlockSpec((_PB, 4), lambda i: (i, 0)),
          _full((128, 512)), _full((128, 512)), _full((1, 512)),
          _full((4, 128)),
      ],
      out_specs=pl.BlockSpec((_PB, 512), lambda i: (i, 0)),
      out_shape=jax.ShapeDtypeStruct((_N4, 512), jnp.float32),
  )(hlo, hhi, agg, agg, deg4, w2lo, w2hi, b2p, s32)


def kernel(x, c, edge_index, W1_0, b1_0, Wc_0, bc_0, W2_0, b2_0,
           W1_1, b1_1, Wc_1, bc_1, W2_1, b2_1):
  i4 = jnp.eye(4, dtype=jnp.float32)
  w1x, w1c = W1_0[:128], W1_0[128:]
  w1a, w1b = W1_1[:_H], W1_1[_H:]

  xp = x.reshape(_N4, 512)
  cp = c.reshape(_N4, 64)
  hlo, hhi, c1p = _tc_pre(
      xp, cp,
      jnp.kron(i4, w1x[:, :_HH]), jnp.kron(i4, w1x[:, _HH:]),
      jnp.kron(i4, w1c[:, :_HH]), jnp.kron(i4, w1c[:, _HH:]),
      jnp.tile(b1_0[:_HH], 4).reshape(1, 128),
      jnp.tile(b1_0[_HH:], 4).reshape(1, 128),
      jnp.kron(i4, Wc_0), jnp.tile(bc_0, 4).reshape(1, 256))

  agg0, deg = _sc_agg_deg(hlo.reshape(_N, _HH), hhi.reshape(_N, _HH),
                          edge_index)
  agg0 = agg0.reshape(2, _PK, 128)
  deg4 = deg.reshape(_N4, 4)
  s32 = jnp.kron(i4, jnp.ones((1, _HH), jnp.float32))
  eye = jnp.eye(_HH, dtype=jnp.float32)
  zero = jnp.zeros((_HH, _HH), jnp.float32)
  sello = jnp.kron(i4, jnp.concatenate([eye, zero], axis=0))
  selhi = jnp.kron(i4, jnp.concatenate([zero, eye], axis=0))

  h1lo, h1hi = _tc_mid(
      hlo, hhi, agg0, deg4, c1p,
      jnp.kron(i4, W2_0[:_HH]), jnp.kron(i4, W2_0[_HH:]),
      jnp.tile(b2_0, 4).reshape(1, 256),
      jnp.kron(i4, w1a), jnp.kron(i4, w1b),
      jnp.tile(b1_1, 4).reshape(1, 256), sello, selhi, s32)

  agg1 = _sc_agg(h1lo.reshape(_N, _HH), h1hi.reshape(_N, _HH), edge_index)
  if isinstance(agg1, (list, tuple)):
    agg1 = agg1[0]
  agg1 = agg1.reshape(2, _PK, 128)
  x2p = _tc_fin(h1lo, h1hi, agg1, deg4,
                jnp.kron(i4, W2_1[:_HH]), jnp.kron(i4, W2_1[_HH:]),
                jnp.tile(b2_1, 4).reshape(1, 512), s32)
  return x2p.reshape(_N, 128)


# speedup vs baseline: 1.6872x; 1.1027x over previous
import functools

import jax
import jax.numpy as jnp
from jax import lax
from jax.experimental import pallas as pl
from jax.experimental.pallas import tpu as pltpu
from jax.experimental.pallas import tpu_sc as plsc

_N = 50000
_E = 800000
_H = 64
_HH = 32
_CHUNK = 128
_TILES = 16
_NCH = _E // _CHUNK
_CPT_A = 391
_CPT_B = 390
_NGRP = 50
_NBUF = 4
_NP = 51200
_ZB = _NP // (_TILES * _CHUNK)
_PK = 12544
_PKT = _PK // _TILES
_N4 = _N // 4
_PB = 512
_GRID = (_N4 + _PB - 1) // _PB


def _sc_aggregate(with_deg):
  mesh = plsc.VectorSubcoreMesh(core_axis_name="c", subcore_axis_name="s")
  out_type = [jax.ShapeDtypeStruct((2, 4 * _PK, _HH), jnp.float32)]
  scratch = [
      pltpu.VMEM((8, _CHUNK), jnp.int32),
      pltpu.VMEM((8, _CHUNK), jnp.int32),
      pltpu.VMEM((_NBUF, _CHUNK, _HH), jnp.float32),
      pltpu.VMEM_SHARED((_NP, _HH), jnp.float32),
      pltpu.SemaphoreType.DMA,
      pltpu.SemaphoreType.DMA,
      pltpu.SemaphoreType.DMA,
  ]
  if with_deg:
    out_type.append(jax.ShapeDtypeStruct((_N,), jnp.float32))
    scratch += [
        pltpu.VMEM((_CHUNK,), jnp.float32),
        pltpu.VMEM((3200,), jnp.float32),
        pltpu.VMEM_SHARED((_NP,), jnp.float32),
        pltpu.SemaphoreType.DMA,
    ]

  def body(h_lo, h_hi, ei, agg_out, *rest):
    if with_deg:
      (deg_out, src_v, dst_v, rows_v, agg_sh, sem_i, sem_g, sem_s,
       ones_v, dzero_v, deg_sh, sem_d) = rest
    else:
      src_v, dst_v, rows_v, agg_sh, sem_i, sem_g, sem_s = rest
    c = lax.axis_index("c")
    s = lax.axis_index("s")

    def _zrow(i, _):
      rows_v[0, i, pl.ds(0, 16)] = jnp.zeros((16,), jnp.float32)
      rows_v[0, i, pl.ds(16, 16)] = jnp.zeros((16,), jnp.float32)
      return 0
    lax.fori_loop(0, _CHUNK, _zrow, 0)

    def _zshared(j, _):
      pltpu.sync_copy(rows_v.at[0],
                      agg_sh.at[pl.ds((s * _ZB + j) * _CHUNK, _CHUNK)])
      return 0
    lax.fori_loop(0, _ZB, _zshared, 0)

    if with_deg:
      def _zd(i, _):
        dzero_v[pl.ds(i * 16, 16)] = jnp.zeros((16,), jnp.float32)
        return 0
      lax.fori_loop(0, 200, _zd, 0)
      pltpu.sync_copy(dzero_v, deg_sh.at[pl.ds(s * 3200, 3200)])

      def _ones(i, _):
        ones_v[pl.ds(i * 16, 16)] = jnp.ones((16,), jnp.float32)
        return 0
      lax.fori_loop(0, _CHUNK // 16, _ones, 0)

    plsc.subcore_barrier()

    cpt = jnp.where(s < 10, _CPT_A, _CPT_B)
    row0 = jnp.where(s < 10, s * _CPT_A, 10 * _CPT_A + (s - 10) * _CPT_B)

    def issue_idx(j, b):
      e0 = (row0 + j) * _CHUNK
      pltpu.async_copy(ei.at[0, pl.ds(e0, _CHUNK)], src_v.at[b], sem_i)
      pltpu.async_copy(ei.at[1, pl.ds(e0, _CHUNK)], dst_v.at[b], sem_i)

    def issue_gather(ib, rb):
      @pl.when(c == 0)
      def _():
        pltpu.async_copy(h_lo.at[src_v.at[ib]], rows_v.at[rb], sem_g)

      @pl.when(c == 1)
      def _():
        pltpu.async_copy(h_hi.at[src_v.at[ib]], rows_v.at[rb], sem_g)

    def wait_idx_pair():
      pltpu.make_async_copy(ei.at[0, pl.ds(0, _CHUNK)], src_v.at[0],
                            sem_i).wait()
      pltpu.make_async_copy(ei.at[0, pl.ds(0, _CHUNK)], dst_v.at[0],
                            sem_i).wait()

    def wait_chunk(sem):
      pltpu.make_async_copy(h_lo.at[pl.ds(0, _CHUNK)], rows_v.at[0],
                            sem).wait()

    if with_deg:
      def wait_deg():
        pltpu.make_async_copy(ones_v, deg_sh.at[pl.ds(0, _CHUNK)],
                              sem_d).wait()

    for b in range(4):
      issue_idx(b, b)
    wait_idx_pair()
    wait_idx_pair()
    issue_gather(0, 0)
    issue_gather(1, 1)

    def grp(g, _):
      for b in range(8):
        j = g * 8 + b

        @pl.when(j + 4 < cpt)
        def _():
          issue_idx(j + 4, (b + 4) % 8)

        @pl.when(jnp.logical_and(j >= 2, j - 2 < cpt))
        def _():
          wait_chunk(sem_s)

        @pl.when(j + 2 < cpt)
        def _():
          wait_idx_pair()
          issue_gather((b + 2) % 8, (b + 2) % _NBUF)

        @pl.when(j < cpt)
        def _():
          wait_chunk(sem_g)
          pltpu.async_copy(rows_v.at[b % _NBUF], agg_sh.at[dst_v.at[b % 8]],
                           sem_s, add=True)

        if with_deg:
          @pl.when(c == 0)
          def _():
            @pl.when(jnp.logical_and(j >= 2, j - 2 < cpt))
            def _():
              wait_deg()

            @pl.when(j < cpt)
            def _():
              pltpu.async_copy(ones_v, deg_sh.at[dst_v.at[b % 8]], sem_d,
                               add=True)
      return 0
    lax.fori_loop(0, _NGRP, grp, 0)

    plsc.subcore_barrier()

    pltpu.sync_copy(agg_sh.at[pl.ds(s * 4 * _PKT, 4 * _PKT)],
                    agg_out.at[c, pl.ds(s * 4 * _PKT, 4 * _PKT)])

    if with_deg:
      @pl.when(jnp.logical_and(c == 0, s == 0))
      def _():
        pltpu.sync_copy(deg_sh.at[pl.ds(0, _N)], deg_out)

  return pl.kernel(
      body, out_type=out_type, mesh=mesh, scratch_types=scratch,
      compiler_params=pltpu.CompilerParams(use_tc_tiling_on_sc=False))


_sc_agg_deg = _sc_aggregate(True)
_sc_agg = _sc_aggregate(False)


def _full(shape):
  return pl.BlockSpec(shape, lambda i: tuple(0 for _ in shape))


def _tc_pre(x, cc, w1x, w1c, b1, wc, bc):
  def body(x_r, c_r, w1x_r, w1c_r, b1_r, wc_r, bc_r, lo_r, hi_r, c1_r):
    los, his, c1s = [], [], []
    for a in range(4):
      xa = x_r[a::4, :]
      ca = c_r[a::4, :]
      h = jnp.maximum(
          jnp.dot(xa, w1x_r[...], preferred_element_type=jnp.float32)
          + jnp.dot(ca, w1c_r[...], preferred_element_type=jnp.float32)
          + b1_r[...], 0.0)
      los.append(h[:, :_HH])
      his.append(h[:, _HH:])
      c1s.append(jnp.maximum(
          jnp.dot(ca, wc_r[...], preferred_element_type=jnp.float32)
          + bc_r[...], 0.0))
    lo_r[...] = jnp.concatenate(los, axis=1)
    hi_r[...] = jnp.concatenate(his, axis=1)
    c1_r[...] = jnp.concatenate(c1s, axis=1)

  pk = pl.BlockSpec((_PB, 128), lambda i: (i, 0))
  return pl.pallas_call(
      body,
      grid=(_GRID,),
      in_specs=[
          pl.BlockSpec((4 * _PB, 128), lambda i: (i, 0)),
          pl.BlockSpec((4 * _PB, 16), lambda i: (i, 0)),
          _full((128, _H)), _full((16, _H)), _full((1, _H)),
          _full((16, _H)), _full((1, _H)),
      ],
      out_specs=[pk, pk, pl.BlockSpec((_PB, 256), lambda i: (i, 0))],
      out_shape=[
          jax.ShapeDtypeStruct((_N4, 128), jnp.float32),
          jax.ShapeDtypeStruct((_N4, 128), jnp.float32),
          jax.ShapeDtypeStruct((_N4, 256), jnp.float32),
      ],
  )(x, cc, w1x, w1c, b1, wc, bc)


def _tc_mid(hlo, hhi, agg, deg4, c1p, w2lo, w2hi, b2p, w1abd, w1bbd, b11p,
            sello, selhi, s32):
  def body(hlo_r, hhi_r, alo_r, ahi_r, deg_r, c1_r, w2lo_r, w2hi_r, b2_r,
           w1a_r, w1b_r, b11_r, sello_r, selhi_r, s32_r, lo_r, hi_r):
    inv4 = 1.0 / jnp.maximum(deg_r[...], 1.0)
    invb = jnp.dot(inv4, s32_r[...], preferred_element_type=jnp.float32)
    plo = hlo_r[...] + alo_r[0] * invb
    phi = hhi_r[...] + ahi_r[0] * invb
    x1 = (jnp.dot(plo, w2lo_r[...], preferred_element_type=jnp.float32)
          + jnp.dot(phi, w2hi_r[...], preferred_element_type=jnp.float32)
          + b2_r[...])
    h1 = jnp.maximum(
        jnp.dot(x1, w1a_r[...], preferred_element_type=jnp.float32)
        + jnp.dot(c1_r[...], w1b_r[...], preferred_element_type=jnp.float32)
        + b11_r[...], 0.0)
    lo_r[...] = jnp.dot(h1, sello_r[...], preferred_element_type=jnp.float32)
    hi_r[...] = jnp.dot(h1, selhi_r[...], preferred_element_type=jnp.float32)

  pk = pl.BlockSpec((_PB, 128), lambda i: (i, 0))
  alo = pl.BlockSpec((1, _PB, 128), lambda i: (0, i, 0))
  ahi = pl.BlockSpec((1, _PB, 128), lambda i: (1, i, 0))
  return pl.pallas_call(
      body,
      grid=(_GRID,),
      in_specs=[
          pk, pk, alo, ahi,
          pl.BlockSpec((_PB, 4), lambda i: (i, 0)),
          pl.BlockSpec((_PB, 256), lambda i: (i, 0)),
          _full((128, 256)), _full((128, 256)), _full((1, 256)),
          _full((256, 256)), _full((256, 256)), _full((1, 256)),
          _full((256, 128)), _full((256, 128)), _full((4, 128)),
      ],
      out_specs=[pk, pk],
      out_shape=[
          jax.ShapeDtypeStruct((_N4, 128), jnp.float32),
          jax.ShapeDtypeStruct((_N4, 128), jnp.float32),
      ],
  )(hlo, hhi, agg, agg, deg4, c1p, w2lo, w2hi, b2p, w1abd, w1bbd, b11p,
    sello, selhi, s32)


def _tc_fin(hlo, hhi, agg, deg4, w2lo, w2hi, b2p, s32):
  def body(hlo_r, hhi_r, alo_r, ahi_r, deg_r, w2lo_r, w2hi_r, b2_r, s32_r,
           out_r):
    inv4 = 1.0 / jnp.maximum(deg_r[...], 1.0)
    invb = jnp.dot(inv4, s32_r[...], preferred_element_type=jnp.float32)
    plo = hlo_r[...] + alo_r[0] * invb
    phi = hhi_r[...] + ahi_r[0] * invb
    out_r[...] = (
        jnp.dot(plo, w2lo_r[...], preferred_element_type=jnp.float32)
        + jnp.dot(phi, w2hi_r[...], preferred_element_type=jnp.float32)
        + b2_r[...])

  pk = pl.BlockSpec((_PB, 128), lambda i: (i, 0))
  alo = pl.BlockSpec((1, _PB, 128), lambda i: (0, i, 0))
  ahi = pl.BlockSpec((1, _PB, 128), lambda i: (1, i, 0))
  return pl.pallas_call(
      body,
      grid=(_GRID,),
      in_specs=[
          pk, pk, alo, ahi,
          pl.BlockSpec((_PB, 4), lambda i: (i, 0)),
          _full((128, 512)), _full((128, 512)), _full((1, 512)),
          _full((4, 128)),
      ],
      out_specs=pl.BlockSpec((_PB, 512), lambda i: (i, 0)),
      out_shape=jax.ShapeDtypeStruct((_N4, 512), jnp.float32),
  )(hlo, hhi, agg, agg, deg4, w2lo, w2hi, b2p, s32)


def kernel(x, c, edge_index, W1_0, b1_0, Wc_0, bc_0, W2_0, b2_0,
           W1_1, b1_1, Wc_1, bc_1, W2_1, b2_1):
  i4 = jnp.eye(4, dtype=jnp.float32)
  w1x, w1c = W1_0[:128], W1_0[128:]
  w1a, w1b = W1_1[:_H], W1_1[_H:]

  hlo, hhi, c1p = _tc_pre(x, c, w1x, w1c, b1_0.reshape(1, _H), Wc_0,
                          bc_0.reshape(1, _H))

  agg0, deg = _sc_agg_deg(hlo.reshape(_N, _HH), hhi.reshape(_N, _HH),
                          edge_index)
  agg0 = agg0.reshape(2, _PK, 128)
  deg4 = deg.reshape(_N4, 4)
  s32 = jnp.kron(i4, jnp.ones((1, _HH), jnp.float32))
  eye = jnp.eye(_HH, dtype=jnp.float32)
  zero = jnp.zeros((_HH, _HH), jnp.float32)
  sello = jnp.kron(i4, jnp.concatenate([eye, zero], axis=0))
  selhi = jnp.kron(i4, jnp.concatenate([zero, eye], axis=0))

  h1lo, h1hi = _tc_mid(
      hlo, hhi, agg0, deg4, c1p,
      jnp.kron(i4, W2_0[:_HH]), jnp.kron(i4, W2_0[_HH:]),
      jnp.tile(b2_0, 4).reshape(1, 256),
      jnp.kron(i4, w1a), jnp.kron(i4, w1b),
      jnp.tile(b1_1, 4).reshape(1, 256), sello, selhi, s32)

  agg1 = _sc_agg(h1lo.reshape(_N, _HH), h1hi.reshape(_N, _HH), edge_index)
  if isinstance(agg1, (list, tuple)):
    agg1 = agg1[0]
  agg1 = agg1.reshape(2, _PK, 128)
  x2p = _tc_fin(h1lo, h1hi, agg1, deg4,
                jnp.kron(i4, W2_1[:_HH]), jnp.kron(i4, W2_1[_HH:]),
                jnp.tile(b2_1, 4).reshape(1, 512), s32)
  return x2p.reshape(_N, 128)


# speedup vs baseline: 1.7885x; 1.0601x over previous
import functools

import jax
import jax.numpy as jnp
from jax import lax
from jax.experimental import pallas as pl
from jax.experimental.pallas import tpu as pltpu
from jax.experimental.pallas import tpu_sc as plsc

_N = 50000
_E = 800000
_H = 64
_HH = 32
_CHUNK = 128
_TILES = 16
_NCH = _E // _CHUNK
_CPT_A = 391
_CPT_B = 390
_NGRP = 50
_NBUF = 4
_NP = 51200
_ZB = _NP // (_TILES * _CHUNK)
_PK = 12544
_PKT = _PK // _TILES
_N4 = _N // 4
_PB = 512
_GRID = (_N4 + _PB - 1) // _PB


def _sc_aggregate(with_deg):
  mesh = plsc.VectorSubcoreMesh(core_axis_name="c", subcore_axis_name="s")
  out_type = [jax.ShapeDtypeStruct((2, 4 * _PK, _HH), jnp.float32)]
  scratch = [
      pltpu.VMEM((8, _CHUNK), jnp.int32),
      pltpu.VMEM((8, _CHUNK), jnp.int32),
      pltpu.VMEM((_NBUF, _CHUNK, _HH), jnp.float32),
      pltpu.VMEM_SHARED((_NP, _HH), jnp.float32),
      pltpu.SemaphoreType.DMA,
      pltpu.SemaphoreType.DMA,
      pltpu.SemaphoreType.DMA,
  ]
  if with_deg:
    out_type.append(jax.ShapeDtypeStruct((_N,), jnp.float32))
    scratch += [
        pltpu.VMEM((_CHUNK,), jnp.float32),
        pltpu.VMEM((3200,), jnp.float32),
        pltpu.VMEM_SHARED((_NP,), jnp.float32),
        pltpu.SemaphoreType.DMA,
    ]

  def body(h_lo, h_hi, ei, agg_out, *rest):
    if with_deg:
      (deg_out, src_v, dst_v, rows_v, agg_sh, sem_i, sem_g, sem_s,
       ones_v, dzero_v, deg_sh, sem_d) = rest
    else:
      src_v, dst_v, rows_v, agg_sh, sem_i, sem_g, sem_s = rest
    c = lax.axis_index("c")
    s = lax.axis_index("s")

    def _zrow(i, _):
      rows_v[0, i, pl.ds(0, 16)] = jnp.zeros((16,), jnp.float32)
      rows_v[0, i, pl.ds(16, 16)] = jnp.zeros((16,), jnp.float32)
      return 0
    lax.fori_loop(0, _CHUNK, _zrow, 0)

    def _zshared(j, _):
      pltpu.sync_copy(rows_v.at[0],
                      agg_sh.at[pl.ds((s * _ZB + j) * _CHUNK, _CHUNK)])
      return 0
    lax.fori_loop(0, _ZB, _zshared, 0)

    if with_deg:
      def _zd(i, _):
        dzero_v[pl.ds(i * 16, 16)] = jnp.zeros((16,), jnp.float32)
        return 0
      lax.fori_loop(0, 200, _zd, 0)
      pltpu.sync_copy(dzero_v, deg_sh.at[pl.ds(s * 3200, 3200)])

      def _ones(i, _):
        ones_v[pl.ds(i * 16, 16)] = jnp.ones((16,), jnp.float32)
        return 0
      lax.fori_loop(0, _CHUNK // 16, _ones, 0)

    plsc.subcore_barrier()

    cpt = jnp.where(s < 10, _CPT_A, _CPT_B)
    row0 = jnp.where(s < 10, s * _CPT_A, 10 * _CPT_A + (s - 10) * _CPT_B)

    def issue_idx(j, b):
      e0 = (row0 + j) * _CHUNK
      pltpu.async_copy(ei.at[0, pl.ds(e0, _CHUNK)], src_v.at[b], sem_i)
      pltpu.async_copy(ei.at[1, pl.ds(e0, _CHUNK)], dst_v.at[b], sem_i)

    def issue_gather(ib, rb):
      @pl.when(c == 0)
      def _():
        pltpu.async_copy(h_lo.at[src_v.at[ib]], rows_v.at[rb], sem_g)

      @pl.when(c == 1)
      def _():
        pltpu.async_copy(h_hi.at[src_v.at[ib]], rows_v.at[rb], sem_g)

    def wait_idx_pair():
      pltpu.make_async_copy(ei.at[0, pl.ds(0, _CHUNK)], src_v.at[0],
                            sem_i).wait()
      pltpu.make_async_copy(ei.at[0, pl.ds(0, _CHUNK)], dst_v.at[0],
                            sem_i).wait()

    def wait_chunk(sem):
      pltpu.make_async_copy(h_lo.at[pl.ds(0, _CHUNK)], rows_v.at[0],
                            sem).wait()

    if with_deg:
      def wait_deg():
        pltpu.make_async_copy(ones_v, deg_sh.at[pl.ds(0, _CHUNK)],
                              sem_d).wait()

    for b in range(4):
      issue_idx(b, b)
    wait_idx_pair()
    wait_idx_pair()
    issue_gather(0, 0)
    issue_gather(1, 1)

    def grp(g, _):
      for b in range(8):
        j = g * 8 + b

        @pl.when(j + 4 < cpt)
        def _():
          issue_idx(j + 4, (b + 4) % 8)

        @pl.when(jnp.logical_and(j >= 2, j - 2 < cpt))
        def _():
          wait_chunk(sem_s)

        @pl.when(j + 2 < cpt)
        def _():
          wait_idx_pair()
          issue_gather((b + 2) % 8, (b + 2) % _NBUF)

        @pl.when(j < cpt)
        def _():
          wait_chunk(sem_g)
          pltpu.async_copy(rows_v.at[b % _NBUF], agg_sh.at[dst_v.at[b % 8]],
                           sem_s, add=True)

        if with_deg:
          @pl.when(c == 0)
          def _():
            @pl.when(jnp.logical_and(j >= 2, j - 2 < cpt))
            def _():
              wait_deg()

            @pl.when(j < cpt)
            def _():
              pltpu.async_copy(ones_v, deg_sh.at[dst_v.at[b % 8]], sem_d,
                               add=True)
      return 0
    lax.fori_loop(0, _NGRP, grp, 0)

    plsc.subcore_barrier()

    pltpu.sync_copy(agg_sh.at[pl.ds(s * 4 * _PKT, 4 * _PKT)],
                    agg_out.at[c, pl.ds(s * 4 * _PKT, 4 * _PKT)])

    if with_deg:
      @pl.when(jnp.logical_and(c == 0, s == 0))
      def _():
        pltpu.sync_copy(deg_sh.at[pl.ds(0, _N)], deg_out)

  return pl.kernel(
      body, out_type=out_type, mesh=mesh, scratch_types=scratch,
      compiler_params=pltpu.CompilerParams(use_tc_tiling_on_sc=False))


_sc_agg_deg = _sc_aggregate(True)
_sc_agg = _sc_aggregate(False)


def _full(shape):
  return pl.BlockSpec(shape, lambda i: tuple(0 for _ in shape))


def _tc_pre(x, cc, w1x, w1c, b1, wc, bc):
  def body(x_r, c_r, w1x_r, w1c_r, b1_r, wc_r, bc_r, lo_r, hi_r, c1_r):
    los, his, c1s = [], [], []
    for a in range(4):
      xa = x_r[a::4, :]
      ca = c_r[a::4, :]
      h = jnp.maximum(
          jnp.dot(xa, w1x_r[...], preferred_element_type=jnp.float32)
          + jnp.dot(ca, w1c_r[...], preferred_element_type=jnp.float32)
          + b1_r[...], 0.0)
      los.append(h[:, :_HH])
      his.append(h[:, _HH:])
      c1s.append(jnp.maximum(
          jnp.dot(ca, wc_r[...], preferred_element_type=jnp.float32)
          + bc_r[...], 0.0))
    lo_r[...] = jnp.concatenate(los, axis=1)
    hi_r[...] = jnp.concatenate(his, axis=1)
    c1_r[...] = jnp.concatenate(c1s, axis=1)

  pk = pl.BlockSpec((_PB, 128), lambda i: (i, 0))
  return pl.pallas_call(
      body,
      grid=(_GRID,),
      in_specs=[
          pl.BlockSpec((4 * _PB, 128), lambda i: (i, 0)),
          pl.BlockSpec((4 * _PB, 16), lambda i: (i, 0)),
          _full((128, _H)), _full((16, _H)), _full((1, _H)),
          _full((16, _H)), _full((1, _H)),
      ],
      out_specs=[pk, pk, pl.BlockSpec((_PB, 256), lambda i: (i, 0))],
      out_shape=[
          jax.ShapeDtypeStruct((_N4, 128), jnp.float32),
          jax.ShapeDtypeStruct((_N4, 128), jnp.float32),
          jax.ShapeDtypeStruct((_N4, 256), jnp.float32),
      ],
  )(x, cc, w1x, w1c, b1, wc, bc)


def _tc_mid(hlo, hhi, agg, deg4, c1p, w2lo, w2hi, b2p, w1abd, w1bbd, b11p,
            sello, selhi, s32):
  def body(hlo_r, hhi_r, alo_r, ahi_r, deg_r, c1_r, w2lo_r, w2hi_r, b2_r,
           w1a_r, w1b_r, b11_r, sello_r, selhi_r, s32_r, lo_r, hi_r):
    inv4 = 1.0 / jnp.maximum(deg_r[...], 1.0)
    invb = jnp.dot(inv4, s32_r[...], preferred_element_type=jnp.float32)
    plo = hlo_r[...] + alo_r[0] * invb
    phi = hhi_r[...] + ahi_r[0] * invb
    x1 = (jnp.dot(plo, w2lo_r[...], preferred_element_type=jnp.float32)
          + jnp.dot(phi, w2hi_r[...], preferred_element_type=jnp.float32)
          + b2_r[...])
    h1 = jnp.maximum(
        jnp.dot(x1, w1a_r[...], preferred_element_type=jnp.float32)
        + jnp.dot(c1_r[...], w1b_r[...], preferred_element_type=jnp.float32)
        + b11_r[...], 0.0)
    lo_r[...] = jnp.dot(h1, sello_r[...], preferred_element_type=jnp.float32)
    hi_r[...] = jnp.dot(h1, selhi_r[...], preferred_element_type=jnp.float32)

  pk = pl.BlockSpec((_PB, 128), lambda i: (i, 0))
  alo = pl.BlockSpec((1, _PB, 128), lambda i: (0, i, 0))
  ahi = pl.BlockSpec((1, _PB, 128), lambda i: (1, i, 0))
  return pl.pallas_call(
      body,
      grid=(_GRID,),
      in_specs=[
          pk, pk, alo, ahi,
          pl.BlockSpec((_PB, 4), lambda i: (i, 0)),
          pl.BlockSpec((_PB, 256), lambda i: (i, 0)),
          _full((128, 256)), _full((128, 256)), _full((1, 256)),
          _full((256, 256)), _full((256, 256)), _full((1, 256)),
          _full((256, 128)), _full((256, 128)), _full((4, 128)),
      ],
      out_specs=[pk, pk],
      out_shape=[
          jax.ShapeDtypeStruct((_N4, 128), jnp.float32),
          jax.ShapeDtypeStruct((_N4, 128), jnp.float32),
      ],
  )(hlo, hhi, agg, agg, deg4, c1p, w2lo, w2hi, b2p, w1abd, w1bbd, b11p,
    sello, selhi, s32)


def _tc_fin(hlo, hhi, agg, deg4, w2, b2, s32):
  def body(hlo_r, hhi_r, alo_r, ahi_r, deg_r, w2_r, b2_r, s32_r, out_r):
    inv4 = 1.0 / jnp.maximum(deg_r[...], 1.0)
    invb = jnp.dot(inv4, s32_r[...], preferred_element_type=jnp.float32)
    plo = hlo_r[...] + alo_r[0] * invb
    phi = hhi_r[...] + ahi_r[0] * invb
    for a in range(4):
      ua = jnp.concatenate(
          [plo[:, a * _HH:(a + 1) * _HH], phi[:, a * _HH:(a + 1) * _HH]],
          axis=1)
      out_r[a::4, :] = (
          jnp.dot(ua, w2_r[...], preferred_element_type=jnp.float32)
          + b2_r[...])

  pk = pl.BlockSpec((_PB, 128), lambda i: (i, 0))
  alo = pl.BlockSpec((1, _PB, 128), lambda i: (0, i, 0))
  ahi = pl.BlockSpec((1, _PB, 128), lambda i: (1, i, 0))
  return pl.pallas_call(
      body,
      grid=(_GRID,),
      in_specs=[
          pk, pk, alo, ahi,
          pl.BlockSpec((_PB, 4), lambda i: (i, 0)),
          _full((_H, 128)), _full((1, 128)), _full((4, 128)),
      ],
      out_specs=pl.BlockSpec((4 * _PB, 128), lambda i: (i, 0)),
      out_shape=jax.ShapeDtypeStruct((_N, 128), jnp.float32),
  )(hlo, hhi, agg, agg, deg4, w2, b2, s32)


def kernel(x, c, edge_index, W1_0, b1_0, Wc_0, bc_0, W2_0, b2_0,
           W1_1, b1_1, Wc_1, bc_1, W2_1, b2_1):
  i4 = jnp.eye(4, dtype=jnp.float32)
  w1x, w1c = W1_0[:128], W1_0[128:]
  w1a, w1b = W1_1[:_H], W1_1[_H:]

  hlo, hhi, c1p = _tc_pre(x, c, w1x, w1c, b1_0.reshape(1, _H), Wc_0,
                          bc_0.reshape(1, _H))

  agg0, deg = _sc_agg_deg(hlo.reshape(_N, _HH), hhi.reshape(_N, _HH),
                          edge_index)
  agg0 = agg0.reshape(2, _PK, 128)
  deg4 = deg.reshape(_N4, 4)
  s32 = jnp.kron(i4, jnp.ones((1, _HH), jnp.float32))
  eye = jnp.eye(_HH, dtype=jnp.float32)
  zero = jnp.zeros((_HH, _HH), jnp.float32)
  sello = jnp.kron(i4, jnp.concatenate([eye, zero], axis=0))
  selhi = jnp.kron(i4, jnp.concatenate([zero, eye], axis=0))

  h1lo, h1hi = _tc_mid(
      hlo, hhi, agg0, deg4, c1p,
      jnp.kron(i4, W2_0[:_HH]), jnp.kron(i4, W2_0[_HH:]),
      jnp.tile(b2_0, 4).reshape(1, 256),
      jnp.kron(i4, w1a), jnp.kron(i4, w1b),
      jnp.tile(b1_1, 4).reshape(1, 256), sello, selhi, s32)

  agg1 = _sc_agg(h1lo.reshape(_N, _HH), h1hi.reshape(_N, _HH), edge_index)
  if isinstance(agg1, (list, tuple)):
    agg1 = agg1[0]
  agg1 = agg1.reshape(2, _PK, 128)
  return _tc_fin(h1lo, h1hi, agg1, deg4, W2_1, b2_1.reshape(1, 128), s32)
